# Initial kernel scaffold; baseline (speedup 1.0000x reference)
import jax
import jax.numpy as jnp
from jax.experimental import pallas as pl


def kernel(x, band_w, band_b, patch_w, patch_b, spectral_prototypes, spatial_prototypes, rw1, rb1, rw2, rb2, wk, bk, n1s, n1b, n2s, n2b, wq, bq, wv, bv, fw1, fb1, fw2, fb2, ns, nb, hw, hb):
    raise NotImplementedError("write your pallas kernel here")


# speedup vs baseline: 5.2660x; 5.2660x over previous
import jax
import jax.numpy as jnp
from jax.experimental import pallas as pl
from jax.experimental.pallas import tpu as pltpu

P = 24
DIM = 192
KS = 32
KP = 32
TOPK = 3
TEMP = 0.1
DEPTH = 4
BR = 24


def _conv_body(x_ref, bw_ref, bb_ref, pw_ref, pb_ref, o_ref):
    _, C, _, W = x_ref.shape
    Wp = W // P
    xb = x_ref[0].reshape(C, P * W)
    yb = jnp.dot(bw_ref[...], xb, preferred_element_type=jnp.float32)
    yb = yb + bb_ref[...]
    t = (yb.astype(jnp.bfloat16).reshape(BR, P, Wp, P)
         .transpose(2, 0, 1, 3).reshape(Wp, BR * P * P))
    out = jnp.dot(t, pw_ref[...], preferred_element_type=jnp.float32) + pb_ref[...]
    o_ref[0, 0] = out


def _gelu(x):
    return 0.5 * x * (1.0 + jax.lax.erf(x * (2.0 ** -0.5)))


def _ln(x, s, b):
    m = x.mean(-1, keepdims=True)
    v = ((x - m) ** 2).mean(-1, keepdims=True)
    return (x - m) / jnp.sqrt(v + 1e-5) * s + b


def _top3_weights(l):
    n, k = l.shape
    iota = jax.lax.broadcasted_iota(jnp.int32, (n, k), 1)
    sel = jnp.zeros(l.shape, dtype=jnp.bool_)
    cur = l
    for _ in range(TOPK):
        m = jnp.max(cur, axis=-1, keepdims=True)
        ismax = cur == m
        idx = jnp.min(jnp.where(ismax, iota, k), axis=-1, keepdims=True)
        pick = iota == idx
        sel = jnp.logical_or(sel, pick)
        cur = jnp.where(pick, -jnp.inf, cur)
    wl = jnp.where(sel, l * (1.0 / TEMP), -jnp.inf)
    return jax.nn.softmax(wl, axis=-1)


def _former_body(t_ref, ps_ref, pp_ref, rw1_ref, rb1_ref, rw2_ref, rb2_ref,
                 wk_ref, bk_ref, n1s_ref, n1b_ref, n2s_ref, n2b_ref,
                 wq_ref, bq_ref, wv_ref, bv_ref, fw1_ref, fb1_ref,
                 fw2_ref, fb2_ref, ns_ref, nb_ref, hw_ref, hb_ref, o_ref):
    x = t_ref[0]
    h = jnp.dot(x, rw1_ref[...], preferred_element_type=jnp.float32) + rb1_ref[...]
    h = _gelu(h)
    logits = jnp.dot(h, rw2_ref[...], preferred_element_type=jnp.float32) + rb2_ref[...]
    ws = _top3_weights(logits[:, :KS])
    wp = _top3_weights(logits[:, KS:])
    spect = jnp.dot(ws, ps_ref[...], preferred_element_type=jnp.float32)
    spat = jnp.dot(wp, pp_ref[...], preferred_element_type=jnp.float32)
    wkm = wk_ref[...]
    keys = (jnp.dot(spect, wkm[:BR], preferred_element_type=jnp.float32)
            + jnp.dot(spat, wkm[BR:], preferred_element_type=jnp.float32)
            + bk_ref[...])
    n1s, n1b = n1s_ref[...], n1b_ref[...]
    n2s, n2b = n2s_ref[...], n2b_ref[...]
    wq, bq = wq_ref[...], bq_ref[...]
    wv, bv = wv_ref[...], bv_ref[...]
    fw1, fb1 = fw1_ref[...], fb1_ref[...]
    fw2, fb2 = fw2_ref[...], fb2_ref[...]
    for i in range(DEPTH):
        hh = _ln(x, n1s[i:i + 1], n1b[i:i + 1])
        q = jnp.dot(hh, wq[i], preferred_element_type=jnp.float32) + bq[i:i + 1]
        v = jnp.dot(hh, wv[i], preferred_element_type=jnp.float32) + bv[i:i + 1]
        kv = jnp.dot(keys.T, v, preferred_element_type=jnp.float32)
        attn = jnp.dot(jax.nn.softmax(q, axis=-1), kv,
                       preferred_element_type=jnp.float32)
        x = x + attn
        h2 = _ln(x, n2s[i:i + 1], n2b[i:i + 1])
        g = _gelu(jnp.dot(h2, fw1[i], preferred_element_type=jnp.float32)
                        + fb1[i:i + 1])
        x = x + jnp.dot(g, fw2[i], preferred_element_type=jnp.float32) + fb2[i:i + 1]
    pooled = jnp.mean(x, axis=0, keepdims=True)
    pooled = _ln(pooled, ns_ref[...], nb_ref[...])
    o_ref[0] = jnp.dot(pooled, hw_ref[...], preferred_element_type=jnp.float32) + hb_ref[...]


def kernel(x, band_w, band_b, patch_w, patch_b, spectral_prototypes,
           spatial_prototypes, rw1, rb1, rw2, rb2, wk, bk, n1s, n1b, n2s, n2b,
           wq, bq, wv, bv, fw1, fb1, fw2, fb2, ns, nb, hw, hb):
    B, C, H, W = x.shape
    Hp, Wp = H // P, W // P
    N = Hp * Wp
    bw2 = band_w.reshape(BR, C)
    bb2 = band_b.reshape(BR, 1)
    pw2 = patch_w.reshape(DIM, BR * P * P).T.astype(jnp.bfloat16)
    pb2 = patch_b.reshape(1, DIM)

    tokens = pl.pallas_call(
        _conv_body,
        grid=(B, Hp),
        in_specs=[
            pl.BlockSpec((1, C, P, W), lambda b, h: (b, 0, h, 0)),
            pl.BlockSpec((BR, C), lambda b, h: (0, 0)),
            pl.BlockSpec((BR, 1), lambda b, h: (0, 0)),
            pl.BlockSpec((BR * P * P, DIM), lambda b, h: (0, 0)),
            pl.BlockSpec((1, DIM), lambda b, h: (0, 0)),
        ],
        out_specs=pl.BlockSpec((1, 1, Wp, DIM), lambda b, h: (b, h, 0, 0)),
        out_shape=jax.ShapeDtypeStruct((B, Hp, Wp, DIM), jnp.float32),
        compiler_params=pltpu.CompilerParams(
            dimension_semantics=("parallel", "arbitrary")),
    )(x, bw2, bb2, pw2, pb2)
    tokens = tokens.reshape(B, N, DIM)

    full = lambda a: pl.BlockSpec(a.shape, lambda b: (0,) * a.ndim)
    weights = [
        spectral_prototypes, spatial_prototypes,
        rw1, rb1.reshape(1, -1), rw2, rb2.reshape(1, -1),
        wk, bk.reshape(1, -1), n1s, n1b, n2s, n2b,
        wq, bq, wv, bv, fw1, fb1, fw2, fb2,
        ns.reshape(1, -1), nb.reshape(1, -1), hw, hb.reshape(1, -1),
    ]
    out = pl.pallas_call(
        _former_body,
        grid=(B,),
        in_specs=[pl.BlockSpec((1, N, DIM), lambda b: (b, 0, 0))]
                 + [full(a) for a in weights],
        out_specs=pl.BlockSpec((1, 1, 16), lambda b: (b, 0, 0)),
        out_shape=jax.ShapeDtypeStruct((B, 1, 16), jnp.float32),
        compiler_params=pltpu.CompilerParams(
            dimension_semantics=("arbitrary",)),
    )(tokens, *weights)
    return out.reshape(B, 16)


# speedup vs baseline: 5.9785x; 1.1353x over previous
import jax
import jax.numpy as jnp
from jax.experimental import pallas as pl
from jax.experimental.pallas import tpu as pltpu

P = 24
DIM = 192
KS = 32
KP = 32
TOPK = 3
TEMP = 0.1
DEPTH = 4
BR = 24


def _conv_body(x_ref, bw_ref, bb_ref, pw_ref, pb_ref, o_ref):
    _, C, _, W = x_ref.shape
    Wp = W // P
    xb = x_ref[0].reshape(C, P * W)
    yb = jnp.dot(bw_ref[...], xb, preferred_element_type=jnp.float32)
    yb = yb + bb_ref[...]
    ybT = yb.astype(jnp.bfloat16).T
    t = (ybT.reshape(P, Wp, P, BR)
         .transpose(1, 0, 2, 3).reshape(Wp, P * P * BR))
    out = jnp.dot(t, pw_ref[...], preferred_element_type=jnp.float32) + pb_ref[...]
    o_ref[0, 0] = out


def _gelu(x):
    return 0.5 * x * (1.0 + jax.lax.erf(x * (2.0 ** -0.5)))


def _ln(x, s, b):
    m = x.mean(-1, keepdims=True)
    v = ((x - m) ** 2).mean(-1, keepdims=True)
    return (x - m) / jnp.sqrt(v + 1e-5) * s + b


def _top3_weights(l):
    n, k = l.shape
    iota = jax.lax.broadcasted_iota(jnp.int32, (n, k), 1)
    sel = jnp.zeros(l.shape, dtype=jnp.bool_)
    cur = l
    for _ in range(TOPK):
        m = jnp.max(cur, axis=-1, keepdims=True)
        ismax = cur == m
        idx = jnp.min(jnp.where(ismax, iota, k), axis=-1, keepdims=True)
        pick = iota == idx
        sel = jnp.logical_or(sel, pick)
        cur = jnp.where(pick, -jnp.inf, cur)
    wl = jnp.where(sel, l * (1.0 / TEMP), -jnp.inf)
    return jax.nn.softmax(wl, axis=-1)


def _former_body(t_ref, ps_ref, pp_ref, rw1_ref, rb1_ref, rw2_ref, rb2_ref,
                 wk_ref, bk_ref, n1s_ref, n1b_ref, n2s_ref, n2b_ref,
                 wq_ref, bq_ref, wv_ref, bv_ref, fw1_ref, fb1_ref,
                 fw2_ref, fb2_ref, ns_ref, nb_ref, hw_ref, hb_ref, o_ref):
    x = t_ref[0]
    h = jnp.dot(x, rw1_ref[...], preferred_element_type=jnp.float32) + rb1_ref[...]
    h = _gelu(h)
    logits = jnp.dot(h, rw2_ref[...], preferred_element_type=jnp.float32) + rb2_ref[...]
    ws = _top3_weights(logits[:, :KS])
    wp = _top3_weights(logits[:, KS:])
    spect = jnp.dot(ws, ps_ref[...], preferred_element_type=jnp.float32)
    spat = jnp.dot(wp, pp_ref[...], preferred_element_type=jnp.float32)
    wkm = wk_ref[...]
    keys = (jnp.dot(spect, wkm[:BR], preferred_element_type=jnp.float32)
            + jnp.dot(spat, wkm[BR:], preferred_element_type=jnp.float32)
            + bk_ref[...])
    n1s, n1b = n1s_ref[...], n1b_ref[...]
    n2s, n2b = n2s_ref[...], n2b_ref[...]
    wq, bq = wq_ref[...], bq_ref[...]
    wv, bv = wv_ref[...], bv_ref[...]
    fw1, fb1 = fw1_ref[...], fb1_ref[...]
    fw2, fb2 = fw2_ref[...], fb2_ref[...]
    for i in range(DEPTH):
        hh = _ln(x, n1s[i:i + 1], n1b[i:i + 1])
        q = jnp.dot(hh, wq[i], preferred_element_type=jnp.float32) + bq[i:i + 1]
        v = jnp.dot(hh, wv[i], preferred_element_type=jnp.float32) + bv[i:i + 1]
        kv = jnp.dot(keys.T, v, preferred_element_type=jnp.float32)
        attn = jnp.dot(jax.nn.softmax(q, axis=-1), kv,
                       preferred_element_type=jnp.float32)
        x = x + attn
        h2 = _ln(x, n2s[i:i + 1], n2b[i:i + 1])
        g = _gelu(jnp.dot(h2, fw1[i], preferred_element_type=jnp.float32)
                        + fb1[i:i + 1])
        x = x + jnp.dot(g, fw2[i], preferred_element_type=jnp.float32) + fb2[i:i + 1]
    pooled = jnp.mean(x, axis=0, keepdims=True)
    pooled = _ln(pooled, ns_ref[...], nb_ref[...])
    o_ref[0] = jnp.dot(pooled, hw_ref[...], preferred_element_type=jnp.float32) + hb_ref[...]


def kernel(x, band_w, band_b, patch_w, patch_b, spectral_prototypes,
           spatial_prototypes, rw1, rb1, rw2, rb2, wk, bk, n1s, n1b, n2s, n2b,
           wq, bq, wv, bv, fw1, fb1, fw2, fb2, ns, nb, hw, hb):
    B, C, H, W = x.shape
    Hp, Wp = H // P, W // P
    N = Hp * Wp
    bw2 = band_w.reshape(BR, C)
    bb2 = band_b.reshape(BR, 1)
    pw2 = patch_w.transpose(2, 3, 1, 0).reshape(BR * P * P, DIM).astype(jnp.bfloat16)
    pb2 = patch_b.reshape(1, DIM)

    tokens = pl.pallas_call(
        _conv_body,
        grid=(B, Hp),
        in_specs=[
            pl.BlockSpec((1, C, P, W), lambda b, h: (b, 0, h, 0)),
            pl.BlockSpec((BR, C), lambda b, h: (0, 0)),
            pl.BlockSpec((BR, 1), lambda b, h: (0, 0)),
            pl.BlockSpec((BR * P * P, DIM), lambda b, h: (0, 0)),
            pl.BlockSpec((1, DIM), lambda b, h: (0, 0)),
        ],
        out_specs=pl.BlockSpec((1, 1, Wp, DIM), lambda b, h: (b, h, 0, 0)),
        out_shape=jax.ShapeDtypeStruct((B, Hp, Wp, DIM), jnp.float32),
        compiler_params=pltpu.CompilerParams(
            dimension_semantics=("parallel", "arbitrary")),
    )(x, bw2, bb2, pw2, pb2)
    tokens = tokens.reshape(B, N, DIM)

    full = lambda a: pl.BlockSpec(a.shape, lambda b: (0,) * a.ndim)
    weights = [
        spectral_prototypes, spatial_prototypes,
        rw1, rb1.reshape(1, -1), rw2, rb2.reshape(1, -1),
        wk, bk.reshape(1, -1), n1s, n1b, n2s, n2b,
        wq, bq, wv, bv, fw1, fb1, fw2, fb2,
        ns.reshape(1, -1), nb.reshape(1, -1), hw, hb.reshape(1, -1),
    ]
    out = pl.pallas_call(
        _former_body,
        grid=(B,),
        in_specs=[pl.BlockSpec((1, N, DIM), lambda b: (b, 0, 0))]
                 + [full(a) for a in weights],
        out_specs=pl.BlockSpec((1, 1, 16), lambda b: (b, 0, 0)),
        out_shape=jax.ShapeDtypeStruct((B, 1, 16), jnp.float32),
        compiler_params=pltpu.CompilerParams(
            dimension_semantics=("arbitrary",)),
    )(tokens, *weights)
    return out.reshape(B, 16)
